# baseline (speedup 1.0000x reference)
import jax
import jax.numpy as jnp
from jax.experimental import pallas as pl
from jax.experimental.pallas import tpu as pltpu


def _mlp_kernel(x_ref, w1_ref, b1_ref, w2_ref, b2_ref, ha_ref, ha_b_ref):
    h = jnp.dot(x_ref[...], w1_ref[...], preferred_element_type=jnp.float32)
    h = jnp.maximum(h + b1_ref[...], 0.0)
    ha = jnp.dot(h, w2_ref[...], preferred_element_type=jnp.float32) + b2_ref[...]
    ha_ref[...] = ha
    ha_b_ref[...] = ha.astype(jnp.bfloat16)


def _spmm_kernel(adj_ref, ha_ref, hp_ref):
    a = adj_ref[...].astype(jnp.bfloat16)
    hp_ref[...] = jnp.dot(a, ha_ref[...], preferred_element_type=jnp.float32)


def kernel(x, adj, w1, b1, w2, b2):
    N, n_in = x.shape
    h1 = w1.shape[1]
    h2 = w2.shape[1]

    b1 = b1.reshape(1, h1)
    b2 = b2.reshape(1, h2)

    tm_mlp = 2048
    while N % tm_mlp:
        tm_mlp //= 2
    ha, ha_b = pl.pallas_call(
        _mlp_kernel,
        out_shape=(
            jax.ShapeDtypeStruct((N, h2), jnp.float32),
            jax.ShapeDtypeStruct((N, h2), jnp.bfloat16),
        ),
        grid=(N // tm_mlp,),
        in_specs=[
            pl.BlockSpec((tm_mlp, n_in), lambda i: (i, 0)),
            pl.BlockSpec((n_in, h1), lambda i: (0, 0)),
            pl.BlockSpec((1, h1), lambda i: (0, 0)),
            pl.BlockSpec((h1, h2), lambda i: (0, 0)),
            pl.BlockSpec((1, h2), lambda i: (0, 0)),
        ],
        out_specs=(
            pl.BlockSpec((tm_mlp, h2), lambda i: (i, 0)),
            pl.BlockSpec((tm_mlp, h2), lambda i: (i, 0)),
        ),
        compiler_params=pltpu.CompilerParams(dimension_semantics=("parallel",)),
    )(x, w1, b1, w2, b2)

    tm = 128
    hp = pl.pallas_call(
        _spmm_kernel,
        out_shape=jax.ShapeDtypeStruct((N, h2), jnp.float32),
        grid=(N // tm,),
        in_specs=[
            pl.BlockSpec((tm, N), lambda i: (i, 0)),
            pl.BlockSpec((N, h2), lambda i: (0, 0)),
        ],
        out_specs=pl.BlockSpec((tm, h2), lambda i: (i, 0)),
        compiler_params=pltpu.CompilerParams(dimension_semantics=("parallel",)),
    )(adj, ha_b)

    return ha, hp


# speedup vs baseline: 1.2124x; 1.2124x over previous
import jax
import jax.numpy as jnp
from jax.experimental import pallas as pl
from jax.experimental.pallas import tpu as pltpu


def _fused_kernel(x_ref, w1_ref, b1_ref, w2_ref, b2_ref, adj_ref,
                  ha_ref, hp_ref, ha_f_scr, ha_b_scr):
    c = pl.program_id(0)
    i = pl.program_id(1)
    ni = pl.num_programs(1)
    tm = adj_ref.shape[0]

    @pl.when(i == 0)
    def _mlp():
        h = jnp.dot(x_ref[...], w1_ref[...], preferred_element_type=jnp.float32)
        h = jnp.maximum(h + b1_ref[...], 0.0)
        ha = jnp.dot(h, w2_ref[...], preferred_element_type=jnp.float32)
        ha = ha + b2_ref[...]
        ha_f_scr[...] = ha
        ha_b_scr[...] = ha.astype(jnp.bfloat16)

    row = (c * ni + i) * tm
    ha_ref[...] = ha_f_scr[pl.ds(row, tm), :]
    a = adj_ref[...].astype(jnp.bfloat16)
    hp_ref[...] = jnp.dot(a, ha_b_scr[...], preferred_element_type=jnp.float32)


def kernel(x, adj, w1, b1, w2, b2):
    N, n_in = x.shape
    h1 = w1.shape[1]
    h2 = w2.shape[1]

    b1 = b1.reshape(1, h1)
    b2 = b2.reshape(1, h2)

    tm = 256
    nblk = N // tm
    ncore = 2 if nblk % 2 == 0 else 1
    ni = nblk // ncore

    ha, hp = pl.pallas_call(
        _fused_kernel,
        out_shape=(
            jax.ShapeDtypeStruct((N, h2), jnp.float32),
            jax.ShapeDtypeStruct((N, h2), jnp.float32),
        ),
        grid=(ncore, ni),
        in_specs=[
            pl.BlockSpec((N, n_in), lambda c, i: (0, 0)),
            pl.BlockSpec((n_in, h1), lambda c, i: (0, 0)),
            pl.BlockSpec((1, h1), lambda c, i: (0, 0)),
            pl.BlockSpec((h1, h2), lambda c, i: (0, 0)),
            pl.BlockSpec((1, h2), lambda c, i: (0, 0)),
            pl.BlockSpec((tm, N), lambda c, i, _ni=ni: (c * _ni + i, 0)),
        ],
        out_specs=(
            pl.BlockSpec((tm, h2), lambda c, i, _ni=ni: (c * _ni + i, 0)),
            pl.BlockSpec((tm, h2), lambda c, i, _ni=ni: (c * _ni + i, 0)),
        ),
        scratch_shapes=[
            pltpu.VMEM((N, h2), jnp.float32),
            pltpu.VMEM((N, h2), jnp.bfloat16),
        ],
        compiler_params=pltpu.CompilerParams(
            dimension_semantics=("parallel", "arbitrary")
        ),
    )(x, w1, b1, w2, b2, adj)

    return ha, hp
